# baseline (speedup 1.0000x reference)
import jax
import jax.numpy as jnp
from jax import lax
from jax.experimental import pallas as pl
from jax.experimental.pallas import tpu as pltpu
from jax.experimental.pallas import tpu_sc as plsc

NC = 2
NS = 16
LANES = 16

CHUNK = 128
D = 64


def _sc_kernel_body(widx_hbm, pidx_hbm, word_hbm, pos_hbm, out_hbm,
                    wi_v, pi_v, pu_v, ph_v,
                    wbuf0, wbuf1, pbuf0, pbuf1,
                    semw0, semw1, semp0, semp1, sems0, sems1):
    wid = lax.axis_index("c") * NS + lax.axis_index("s")
    n_per_tile = widx_hbm.shape[0] // (NC * NS)
    n_chunks = n_per_tile // CHUNK
    base = wid * n_per_tile

    pltpu.sync_copy(widx_hbm.at[pl.ds(base, n_per_tile)],
                    wi_v.at[pl.ds(0, n_per_tile)])
    pltpu.sync_copy(pidx_hbm.at[pl.ds(base, n_per_tile)],
                    pi_v.at[pl.ds(0, n_per_tile)])

    def idx_body(i, _):
        sl = pl.ds(i * LANES, LANES)
        p = pi_v[sl]
        pu_v[sl] = lax.shift_right_logical(p, 1)
        ph_v[sl] = lax.shift_left(lax.bitwise_and(p, 1), 6)
        return 0

    lax.fori_loop(0, n_per_tile // LANES, idx_body, 0, unroll=False)

    wbufs = [wbuf0, wbuf1]
    pbufs = [pbuf0, pbuf1]
    semws = [semw0, semw1]
    semps = [semp0, semp1]
    semss = [sems0, sems1]
    cp = [None, None]
    cs = [None, None]

    def issue_chunk(k, b):
        koff = k * CHUNK
        wbuf = wbufs[b]
        semw = semws[b]
        cp[b] = pltpu.async_copy(
            pos_hbm.at[pu_v.at[pl.ds(koff, CHUNK)]], pbufs[b], semps[b])

        def issue_body(g, _):
            iv = wi_v[pl.ds(koff + g * LANES, LANES)]
            for j in range(LANES):
                pltpu.async_copy(word_hbm.at[iv[j]],
                                 wbuf.at[g * LANES + j], semw)
            return 0

        lax.fori_loop(0, CHUNK // LANES, issue_body, 0, unroll=False)

    def drain_chunk(b):
        wbuf = wbufs[b]
        semw = semws[b]

        def drain_body(r, _):
            pltpu.make_async_copy(word_hbm.at[0], wbuf.at[r], semw).wait()
            return 0

        lax.fori_loop(0, CHUNK, drain_body, 0, unroll=False)
        cp[b].wait()

    issue_chunk(0, 0)

    for k in range(n_chunks):
        b = k % 2
        nb = (k + 1) % 2
        if k + 1 < n_chunks:
            if cs[nb] is not None:
                cs[nb].wait()
            issue_chunk(k + 1, nb)
        drain_chunk(b)

        wbuf, pbuf = wbufs[b], pbufs[b]
        koff = k * CHUNK

        def add_body(r, _):
            hp = ph_v[pl.ds(koff + r, LANES)][0]
            for c in range(D // LANES):
                sl = pl.ds(c * LANES, LANES)
                wbuf[r, sl] = wbuf[r, sl] + pbuf[r, pl.ds(hp + c * LANES,
                                                          LANES)]
            return 0

        lax.fori_loop(0, CHUNK, add_body, 0, unroll=False)
        cs[b] = pltpu.async_copy(
            wbuf, out_hbm.at[pl.ds(base + koff, CHUNK)], semss[b])

    for c in cs:
        if c is not None:
            c.wait()


def kernel(src_word, src_pos, word_emb, pos_emb):
    B, S = src_word.shape
    n = B * S
    P, _ = pos_emb.shape
    widx = src_word.reshape(n)
    pidx = src_pos.reshape(n)
    p2 = pos_emb.reshape(P // 2, 2 * D)

    mesh = plsc.VectorSubcoreMesh(core_axis_name="c", subcore_axis_name="s",
                                  num_cores=NC, num_subcores=NS)
    n_per_tile = n // (NC * NS)
    run = pl.kernel(
        _sc_kernel_body,
        out_type=jax.ShapeDtypeStruct((n, D), jnp.float32),
        mesh=mesh,
        compiler_params=pltpu.CompilerParams(use_tc_tiling_on_sc=True),
        scratch_types=[
            pltpu.VMEM((n_per_tile + LANES,), jnp.int32),
            pltpu.VMEM((n_per_tile + LANES,), jnp.int32),
            pltpu.VMEM((n_per_tile,), jnp.int32),
            pltpu.VMEM((n_per_tile + LANES,), jnp.int32),
            pltpu.VMEM((CHUNK, D), jnp.float32),
            pltpu.VMEM((CHUNK, D), jnp.float32),
            pltpu.VMEM((CHUNK, 2 * D), jnp.float32),
            pltpu.VMEM((CHUNK, 2 * D), jnp.float32),
            pltpu.SemaphoreType.DMA,
            pltpu.SemaphoreType.DMA,
            pltpu.SemaphoreType.DMA,
            pltpu.SemaphoreType.DMA,
            pltpu.SemaphoreType.DMA,
            pltpu.SemaphoreType.DMA,
        ],
    )
    out = run(widx, pidx, word_emb, p2)
    return out.reshape(B, S, D)
